# Initial kernel scaffold; baseline (speedup 1.0000x reference)
import jax
import jax.numpy as jnp
from jax.experimental import pallas as pl


def kernel(input_ids, table):
    raise NotImplementedError("write your pallas kernel here")


# speedup vs baseline: 3.0686x; 3.0686x over previous
import functools

import jax
import jax.numpy as jnp
from jax import lax
from jax.experimental import pallas as pl
from jax.experimental.pallas import tpu as pltpu
from jax.experimental.pallas import tpu_sc as plsc

B, S, D = 4096, 50, 128
N = B * S
NC, NS = 2, 16
NW = NC * NS
PER_W = N // NW
CHUNK = 128
NCH = PER_W // CHUNK


def _make_emb_kernel():
    mesh = plsc.VectorSubcoreMesh(core_axis_name="c", subcore_axis_name="s")

    @functools.partial(
        pl.kernel,
        mesh=mesh,
        out_type=jax.ShapeDtypeStruct((N, D), jnp.float32),
        scratch_types=[
            pltpu.VMEM((NCH, CHUNK), jnp.int32),
            pltpu.VMEM((CHUNK, D), jnp.float32),
            pltpu.SemaphoreType.DMA,
        ],
    )
    def emb(idx_hbm, table_hbm, out_hbm, idx_v, rows_v, gsem):
        wid = lax.axis_index("s") * NC + lax.axis_index("c")
        pltpu.sync_copy(idx_hbm.at[wid], idx_v)
        out_base = wid * PER_W

        def step(j, carry):
            pltpu.async_copy(table_hbm.at[idx_v.at[j]], rows_v, gsem).wait()
            pltpu.sync_copy(rows_v, out_hbm.at[pl.ds(out_base + j * CHUNK, CHUNK)])
            return carry

        lax.fori_loop(0, NCH, step, 0)

    return emb


_emb = _make_emb_kernel()


def kernel(input_ids, table):
    idx = input_ids.reshape(NW, NCH, CHUNK).astype(jnp.int32)
    out = _emb(idx, table)
    return out.reshape(B, S, D)


# speedup vs baseline: 6.1696x; 2.0106x over previous
import functools

import jax
import jax.numpy as jnp
from jax import lax
from jax.experimental import pallas as pl
from jax.experimental.pallas import tpu as pltpu
from jax.experimental.pallas import tpu_sc as plsc

B, S, D = 4096, 50, 128
NC, NS = 2, 16
NW = NC * NS
SEQ_W = B // NW
CSEQ = 1
NCH = SEQ_W // CSEQ
NB = 8
SLACK = 3


def _make_emb_kernel():
    mesh = plsc.VectorSubcoreMesh(core_axis_name="c", subcore_axis_name="s")

    @functools.partial(
        pl.kernel,
        mesh=mesh,
        out_type=jax.ShapeDtypeStruct((B, S, D), jnp.float32),
        scratch_types=(
            [pltpu.VMEM((SEQ_W, S), jnp.int32)]
            + [pltpu.VMEM((S, D), jnp.float32) for _ in range(NB)]
            + [pltpu.SemaphoreType.DMA for _ in range(2 * NB)]
        ),
    )
    def emb(idx_hbm, table_hbm, out_hbm, idx_v, *rest):
        bufs = rest[:NB]
        gsem = rest[NB:2 * NB]
        ssem = rest[2 * NB:]
        wid = lax.axis_index("s") * NC + lax.axis_index("c")
        pltpu.sync_copy(idx_hbm.at[wid], idx_v)
        seq_base = wid * SEQ_W

        def gather_copy(j, b):
            return pltpu.make_async_copy(
                table_hbm.at[idx_v.at[j]], bufs[b], gsem[b])

        def store_copy(j, b):
            return pltpu.make_async_copy(
                bufs[b], out_hbm.at[seq_base + j], ssem[b])

        def step(j, i, mid):
            gather_copy(j, i).wait()
            store_copy(j, i).start()
            if mid:
                bp = (i - SLACK) % NB
                store_copy(j - SLACK, bp).wait()
                gather_copy(j + NB - SLACK, bp).start()

        for jj in range(NB):
            gather_copy(jj, jj).start()
        for i in range(NB):
            step(i, i, i >= SLACK)

        def group(g, c):
            j0 = g * NB
            for i in range(NB):
                step(j0 + i, i, True)
            return c

        lax.fori_loop(1, NCH // NB - 1, group, 0)

        j0 = NCH - NB
        for i in range(NB):
            step(j0 + i, i, i < SLACK)
        for i in range(NB):
            store_copy(j0 + i, i).wait()

    return emb


_emb = _make_emb_kernel()


def kernel(input_ids, table):
    idx = input_ids.reshape(NW, SEQ_W, S).astype(jnp.int32)
    return _emb(idx, table)
